# baseline (speedup 1.0000x reference)
import jax
import jax.numpy as jnp
from jax import lax
from jax.experimental import pallas as pl
from jax.experimental.pallas import tpu as pltpu

_B, _T, _D, _V = 32, 8, 1024, 100000
_BT = _B * _T
_W_BLK = 2048
_O_MULT = 4
_O_BLK = _W_BLK * _O_MULT


def _proj_kernel(x_ref, w_ref, b_ref, o_ref):
    j = pl.program_id(0)
    xb = x_ref[...].astype(jnp.bfloat16)
    wb = w_ref[...].astype(jnp.bfloat16)
    acc = jax.lax.dot_general(
        xb, wb, (((1,), (1,)), ((), ())), preferred_element_type=jnp.float32
    )
    col = lax.rem(j, _O_MULT) * _W_BLK
    o_ref[:, pl.ds(col, _W_BLK)] = acc + b_ref[...]


def kernel(x, W, b):
    x2 = x.reshape(_BT, _D)
    b2 = b.reshape(1, _V)
    grid = (pl.cdiv(_V, _W_BLK),)
    out = pl.pallas_call(
        _proj_kernel,
        grid=grid,
        in_specs=[
            pl.BlockSpec((_BT, _D), lambda j: (0, 0)),
            pl.BlockSpec((_W_BLK, _D), lambda j: (j, 0)),
            pl.BlockSpec((1, _W_BLK), lambda j: (0, j)),
        ],
        out_specs=pl.BlockSpec((_BT, _O_BLK), lambda j: (0, j // _O_MULT)),
        out_shape=jax.ShapeDtypeStruct((_BT, _V), jnp.float32),
        compiler_params=pltpu.CompilerParams(
            dimension_semantics=("arbitrary",),
        ),
    )(x2, W, b2)
    return out.reshape(_B, _T, _V)


# speedup vs baseline: 1.0248x; 1.0248x over previous
import jax
import jax.numpy as jnp
from jax.experimental import pallas as pl
from jax.experimental.pallas import tpu as pltpu

_B, _T, _D, _V = 32, 8, 1024, 100000
_BT = _B * _T
_V_BLK = 3072


def _proj_kernel(x_ref, w_ref, b_ref, o_ref):
    xb = x_ref[...].astype(jnp.bfloat16)
    wb = w_ref[...].astype(jnp.bfloat16)
    acc = jax.lax.dot_general(
        xb, wb, (((1,), (1,)), ((), ())), preferred_element_type=jnp.float32
    )
    o_ref[...] = acc + b_ref[...]


def kernel(x, W, b):
    x2 = x.reshape(_BT, _D)
    b2 = b.reshape(1, _V)
    grid = (pl.cdiv(_V, _V_BLK),)
    out = pl.pallas_call(
        _proj_kernel,
        grid=grid,
        in_specs=[
            pl.BlockSpec((_BT, _D), lambda j: (0, 0)),
            pl.BlockSpec((_V_BLK, _D), lambda j: (j, 0)),
            pl.BlockSpec((1, _V_BLK), lambda j: (0, j)),
        ],
        out_specs=pl.BlockSpec((_BT, _V_BLK), lambda j: (0, j)),
        out_shape=jax.ShapeDtypeStruct((_BT, _V), jnp.float32),
        compiler_params=pltpu.CompilerParams(
            dimension_semantics=("arbitrary",),
        ),
    )(x2, W, b2)
    return out.reshape(_B, _T, _V)
